# baseline (speedup 1.0000x reference)
import functools

import jax
import jax.numpy as jnp
from jax import lax
from jax.experimental import pallas as pl
from jax.experimental.pallas import tpu as pltpu
from jax.experimental.pallas import tpu_sc as plsc

BATCH = 16384
KNOW = 128
EXER_N = 100000
NC, NS = 2, 16
NW = NC * NS
BPW = BATCH // NW
CH = 128
NCH = BPW // CH
T_INV = 1.0 / 50.0
TC_ROWS = 2048


def _gather_body(user_hbm, item_hbm, theta_hbm, sg_hbm,
                 thg_hbm, sraw_hbm, graw_hbm,
                 idx_u, idx_i, idx_g, t0, t1, s0, s1, g0, g1,
                 sem_i0, sem_i1, sem_o0, sem_o1):
    wid = lax.axis_index("s") * NC + lax.axis_index("c")
    base = wid * BPW

    pltpu.sync_copy(user_hbm.at[pl.ds(base, BPW)], idx_u)
    pltpu.sync_copy(item_hbm.at[pl.ds(base, BPW)], idx_i)

    def mk_guess_idx(j, carry):
        sl = pl.ds(j * 16, 16)
        idx_g[sl] = idx_i[sl] + EXER_N
        return carry

    lax.fori_loop(0, BPW // 16, mk_guess_idx, 0)

    tb, sb, gb = (t0, t1), (s0, s1), (g0, g1)
    sem_i, sem_o = (sem_i0, sem_i1), (sem_o0, sem_o1)

    def fire_in(ch):
        slot = ch % 2
        ids = pl.ds(ch * CH, CH)
        return (
            pltpu.async_copy(theta_hbm.at[idx_u.at[ids]], tb[slot], sem_i[slot]),
            pltpu.async_copy(sg_hbm.at[idx_i.at[ids]], sb[slot], sem_i[slot]),
            pltpu.async_copy(sg_hbm.at[idx_g.at[ids]], gb[slot], sem_i[slot]),
        )

    def fire_out(ch):
        slot = ch % 2
        ids = pl.ds(base + ch * CH, CH)
        return (
            pltpu.async_copy(tb[slot], thg_hbm.at[ids], sem_o[slot]),
            pltpu.async_copy(sb[slot], sraw_hbm.at[ids], sem_o[slot]),
            pltpu.async_copy(gb[slot], graw_hbm.at[ids], sem_o[slot]),
        )

    pend_in = {0: fire_in(0), 1: fire_in(1)}
    tail = []
    for ch in range(NCH):
        for c in pend_in[ch]:
            c.wait()
        out_cp = fire_out(ch)
        if ch + 2 < NCH:
            for c in out_cp:
                c.wait()
            pend_in[ch + 2] = fire_in(ch + 2)
        else:
            tail.append(out_cp)
    for out_cp in tail:
        for c in out_cp:
            c.wait()


@jax.jit
def _sc_gather(user, item, theta_table, sg_flat):
    run = pl.kernel(
        _gather_body,
        out_type=(
            jax.ShapeDtypeStruct((BATCH, KNOW), jnp.float32),
            jax.ShapeDtypeStruct((BATCH,), jnp.float32),
            jax.ShapeDtypeStruct((BATCH,), jnp.float32),
        ),
        mesh=plsc.VectorSubcoreMesh(core_axis_name="c", subcore_axis_name="s",
                                    num_cores=NC, num_subcores=NS),
        compiler_params=pltpu.CompilerParams(needs_layout_passes=False),
        scratch_types=[
            pltpu.VMEM((BPW,), jnp.int32),
            pltpu.VMEM((BPW,), jnp.int32),
            pltpu.VMEM((BPW,), jnp.int32),
            pltpu.VMEM((CH, KNOW), jnp.float32),
            pltpu.VMEM((CH, KNOW), jnp.float32),
            pltpu.VMEM((CH,), jnp.float32),
            pltpu.VMEM((CH,), jnp.float32),
            pltpu.VMEM((CH,), jnp.float32),
            pltpu.VMEM((CH,), jnp.float32),
            pltpu.SemaphoreType.DMA,
            pltpu.SemaphoreType.DMA,
            pltpu.SemaphoreType.DMA,
            pltpu.SemaphoreType.DMA,
        ],
        name="dina_sc_gather",
    )
    return run(user, item, theta_table, sg_flat)


def _combine_body(th_ref, kn_ref, sr_ref, gr_ref, o_ref):
    th = th_ref[...]
    kn = kn_ref[...]
    s = 0.5 * jnp.tanh(0.5 * th)
    prod_t = lax.transpose(kn * s, (1, 0))
    n = jnp.sum(prod_t, axis=0)
    p = 0.5 * jnp.tanh((0.5 * T_INV) * n) + 0.5
    slip = 0.2 * jnp.tanh(0.5 * sr_ref[...]) + 0.2
    guess = 0.2 * jnp.tanh(0.5 * gr_ref[...]) + 0.2
    o_ref[...] = guess + (1.0 - slip - guess) * p


@jax.jit
def _tc_combine(theta_g, knowledge, s_raw, g_raw):
    return pl.pallas_call(
        _combine_body,
        grid=(BATCH // TC_ROWS,),
        in_specs=[
            pl.BlockSpec((TC_ROWS, KNOW), lambda i: (i, 0)),
            pl.BlockSpec((TC_ROWS, KNOW), lambda i: (i, 0)),
            pl.BlockSpec((TC_ROWS,), lambda i: (i,)),
            pl.BlockSpec((TC_ROWS,), lambda i: (i,)),
        ],
        out_specs=pl.BlockSpec((TC_ROWS,), lambda i: (i,)),
        out_shape=jax.ShapeDtypeStruct((BATCH,), jnp.float32),
        name="dina_tc_combine",
    )(theta_g, knowledge, s_raw, g_raw)


def kernel(user, item, knowledge, theta_table, slip_table, guess_table):
    sg_flat = jnp.concatenate([slip_table, guess_table]).reshape(-1)
    theta_g, s_raw, g_raw = _sc_gather(user, item, theta_table, sg_flat)
    return _tc_combine(theta_g, knowledge, s_raw, g_raw)


# speedup vs baseline: 1.0118x; 1.0118x over previous
import functools

import jax
import jax.numpy as jnp
from jax import lax
from jax.experimental import pallas as pl
from jax.experimental.pallas import tpu as pltpu
from jax.experimental.pallas import tpu_sc as plsc

BATCH = 16384
KNOW = 128
NC, NS = 2, 16
NW = NC * NS
CH = 128
T_INV = 1.0 / 50.0
TC_ROWS = 2048
HALF = BATCH // 2


def _gather_body(user_hbm, item_hbm, theta_hbm, slip_hbm, guess_hbm,
                 thg_hbm, sraw_hbm, graw_hbm,
                 idx_u, idx_i, t0, t1, s0, s1, g0, g1,
                 sem_i0, sem_i1, sem_o0, sem_o1):
    bpw = HALF // NW
    nch = bpw // CH
    wid = lax.axis_index("s") * NC + lax.axis_index("c")
    base = wid * bpw

    pltpu.sync_copy(user_hbm.at[pl.ds(base, bpw)], idx_u)
    pltpu.sync_copy(item_hbm.at[pl.ds(base, bpw)], idx_i)

    tb, sb, gb = (t0, t1), (s0, s1), (g0, g1)
    sem_i, sem_o = (sem_i0, sem_i1), (sem_o0, sem_o1)

    def fire_in(ch):
        slot = ch % 2
        ids = pl.ds(ch * CH, CH)
        return (
            pltpu.async_copy(theta_hbm.at[idx_u.at[ids]], tb[slot], sem_i[slot]),
            pltpu.async_copy(slip_hbm.at[idx_i.at[ids]], sb[slot], sem_i[slot]),
            pltpu.async_copy(guess_hbm.at[idx_i.at[ids]], gb[slot], sem_i[slot]),
        )

    def fire_out(ch):
        slot = ch % 2
        ids = pl.ds(base + ch * CH, CH)
        return (
            pltpu.async_copy(tb[slot], thg_hbm.at[ids], sem_o[slot]),
            pltpu.async_copy(sb[slot], sraw_hbm.at[ids], sem_o[slot]),
            pltpu.async_copy(gb[slot], graw_hbm.at[ids], sem_o[slot]),
        )

    pend_in = {0: fire_in(0), 1: fire_in(1)}
    tail = []
    for ch in range(nch):
        for c in pend_in[ch]:
            c.wait()
        out_cp = fire_out(ch)
        if ch + 2 < nch:
            for c in out_cp:
                c.wait()
            pend_in[ch + 2] = fire_in(ch + 2)
        else:
            tail.append(out_cp)
    for out_cp in tail:
        for c in out_cp:
            c.wait()


@jax.jit
def _sc_gather(user, item, theta_table, slip_flat, guess_flat):
    run = pl.kernel(
        _gather_body,
        out_type=(
            jax.ShapeDtypeStruct((HALF, KNOW), jnp.float32),
            jax.ShapeDtypeStruct((HALF,), jnp.float32),
            jax.ShapeDtypeStruct((HALF,), jnp.float32),
        ),
        mesh=plsc.VectorSubcoreMesh(core_axis_name="c", subcore_axis_name="s",
                                    num_cores=NC, num_subcores=NS),
        compiler_params=pltpu.CompilerParams(needs_layout_passes=False),
        scratch_types=[
            pltpu.VMEM((HALF // NW,), jnp.int32),
            pltpu.VMEM((HALF // NW,), jnp.int32),
            pltpu.VMEM((CH, KNOW), jnp.float32),
            pltpu.VMEM((CH, KNOW), jnp.float32),
            pltpu.VMEM((CH,), jnp.float32),
            pltpu.VMEM((CH,), jnp.float32),
            pltpu.VMEM((CH,), jnp.float32),
            pltpu.VMEM((CH,), jnp.float32),
            pltpu.SemaphoreType.DMA,
            pltpu.SemaphoreType.DMA,
            pltpu.SemaphoreType.DMA,
            pltpu.SemaphoreType.DMA,
        ],
        name="dina_sc_gather",
    )
    return run(user, item, theta_table, slip_flat, guess_flat)


def _combine_body(th_ref, kn_ref, sr_ref, gr_ref, o_ref):
    th = th_ref[...]
    kn = kn_ref[...]
    s = 0.5 * jnp.tanh(0.5 * th)
    prod_t = lax.transpose(kn * s, (1, 0))
    n = jnp.sum(prod_t, axis=0)
    p = 0.5 * jnp.tanh((0.5 * T_INV) * n) + 0.5
    slip = 0.2 * jnp.tanh(0.5 * sr_ref[...]) + 0.2
    guess = 0.2 * jnp.tanh(0.5 * gr_ref[...]) + 0.2
    o_ref[...] = guess + (1.0 - slip - guess) * p


@functools.partial(jax.jit, static_argnames=("kn_off",))
def _tc_combine(theta_g, knowledge, s_raw, g_raw, kn_off):
    return pl.pallas_call(
        _combine_body,
        grid=(HALF // TC_ROWS,),
        in_specs=[
            pl.BlockSpec((TC_ROWS, KNOW), lambda i: (i, 0)),
            pl.BlockSpec((TC_ROWS, KNOW), lambda i, o=kn_off: (i + o, 0)),
            pl.BlockSpec((TC_ROWS,), lambda i: (i,)),
            pl.BlockSpec((TC_ROWS,), lambda i: (i,)),
        ],
        out_specs=pl.BlockSpec((TC_ROWS,), lambda i: (i,)),
        out_shape=jax.ShapeDtypeStruct((HALF,), jnp.float32),
        name="dina_tc_combine",
    )(theta_g, knowledge, s_raw, g_raw)


def kernel(user, item, knowledge, theta_table, slip_table, guess_table):
    slip_flat = slip_table.reshape(-1)
    guess_flat = guess_table.reshape(-1)
    th1, s1, g1 = _sc_gather(user[:HALF], item[:HALF], theta_table,
                             slip_flat, guess_flat)
    th2, s2, g2 = _sc_gather(user[HALF:], item[HALF:], theta_table,
                             slip_flat, guess_flat)
    o1 = _tc_combine(th1, knowledge, s1, g1, 0)
    o2 = _tc_combine(th2, knowledge, s2, g2, HALF // TC_ROWS)
    return jnp.concatenate([o1, o2])


# speedup vs baseline: 1.1410x; 1.1277x over previous
import functools

import jax
import jax.numpy as jnp
from jax import lax
from jax.experimental import pallas as pl
from jax.experimental.pallas import tpu as pltpu
from jax.experimental.pallas import tpu_sc as plsc

BATCH = 16384
KNOW = 128
NC, NS = 2, 16
NW = NC * NS
BPW = BATCH // NW
CH = 128
NCH = BPW // CH
T_INV = 1.0 / 50.0
TC_ROWS = 2048


def _gather_body(user_hbm, item_hbm, theta_hbm, slip_hbm, guess_hbm,
                 thg_hbm, sraw_hbm, graw_hbm,
                 idx_u, idx_i, t0, t1, t2, t3, s0, s1, s2, s3,
                 g0, g1, g2, g3, sem_i0, sem_i1, sem_i2, sem_i3, sem_o):
    wid = lax.axis_index("s") * NC + lax.axis_index("c")
    base = wid * BPW

    pltpu.sync_copy(user_hbm.at[pl.ds(base, BPW)], idx_u)
    pltpu.sync_copy(item_hbm.at[pl.ds(base, BPW)], idx_i)

    tb, sb, gb = (t0, t1, t2, t3), (s0, s1, s2, s3), (g0, g1, g2, g3)
    sem_i = (sem_i0, sem_i1, sem_i2, sem_i3)

    def fire_in(ch):
        ids = pl.ds(ch * CH, CH)
        return (
            pltpu.async_copy(theta_hbm.at[idx_u.at[ids]], tb[ch], sem_i[ch]),
            pltpu.async_copy(slip_hbm.at[idx_i.at[ids]], sb[ch], sem_i[ch]),
            pltpu.async_copy(guess_hbm.at[idx_i.at[ids]], gb[ch], sem_i[ch]),
        )

    def fire_out(ch):
        ids = pl.ds(base + ch * CH, CH)
        return (
            pltpu.async_copy(tb[ch], thg_hbm.at[ids], sem_o),
            pltpu.async_copy(sb[ch], sraw_hbm.at[ids], sem_o),
            pltpu.async_copy(gb[ch], graw_hbm.at[ids], sem_o),
        )

    pend_in = [fire_in(ch) for ch in range(NCH)]
    pend_out = []
    for ch in range(NCH):
        for c in pend_in[ch]:
            c.wait()
        pend_out.append(fire_out(ch))
    for out_cp in pend_out:
        for c in out_cp:
            c.wait()


@jax.jit
def _sc_gather(user, item, theta_table, slip_flat, guess_flat):
    run = pl.kernel(
        _gather_body,
        out_type=(
            jax.ShapeDtypeStruct((BATCH, KNOW), jnp.float32),
            jax.ShapeDtypeStruct((BATCH,), jnp.float32),
            jax.ShapeDtypeStruct((BATCH,), jnp.float32),
        ),
        mesh=plsc.VectorSubcoreMesh(core_axis_name="c", subcore_axis_name="s",
                                    num_cores=NC, num_subcores=NS),
        compiler_params=pltpu.CompilerParams(needs_layout_passes=False),
        scratch_types=(
            [pltpu.VMEM((BPW,), jnp.int32)] * 2
            + [pltpu.VMEM((CH, KNOW), jnp.float32)] * NCH
            + [pltpu.VMEM((CH,), jnp.float32)] * NCH
            + [pltpu.VMEM((CH,), jnp.float32)] * NCH
            + [pltpu.SemaphoreType.DMA] * (NCH + 1)
        ),
        name="dina_sc_gather",
    )
    return run(user, item, theta_table, slip_flat, guess_flat)


def _combine_body(th_ref, kn_ref, sr_ref, gr_ref, o_ref):
    th = th_ref[...]
    kn = kn_ref[...]
    s = 0.5 * jnp.tanh(0.5 * th)
    prod_t = lax.transpose(kn * s, (1, 0))
    n = jnp.sum(prod_t, axis=0)
    p = 0.5 * jnp.tanh((0.5 * T_INV) * n) + 0.5
    slip = 0.2 * jnp.tanh(0.5 * sr_ref[...]) + 0.2
    guess = 0.2 * jnp.tanh(0.5 * gr_ref[...]) + 0.2
    o_ref[...] = guess + (1.0 - slip - guess) * p


@jax.jit
def _tc_combine(theta_g, knowledge, s_raw, g_raw):
    return pl.pallas_call(
        _combine_body,
        grid=(BATCH // TC_ROWS,),
        in_specs=[
            pl.BlockSpec((TC_ROWS, KNOW), lambda i: (i, 0)),
            pl.BlockSpec((TC_ROWS, KNOW), lambda i: (i, 0)),
            pl.BlockSpec((TC_ROWS,), lambda i: (i,)),
            pl.BlockSpec((TC_ROWS,), lambda i: (i,)),
        ],
        out_specs=pl.BlockSpec((TC_ROWS,), lambda i: (i,)),
        out_shape=jax.ShapeDtypeStruct((BATCH,), jnp.float32),
        name="dina_tc_combine",
    )(theta_g, knowledge, s_raw, g_raw)


def kernel(user, item, knowledge, theta_table, slip_table, guess_table):
    theta_g, s_raw, g_raw = _sc_gather(
        user, item, theta_table,
        slip_table.reshape(-1), guess_table.reshape(-1))
    return _tc_combine(theta_g, knowledge, s_raw, g_raw)


# speedup vs baseline: 1.1958x; 1.0481x over previous
import functools

import jax
import jax.numpy as jnp
from jax import lax
from jax.experimental import pallas as pl
from jax.experimental.pallas import tpu as pltpu
from jax.experimental.pallas import tpu_sc as plsc

BATCH = 16384
KNOW = 128
NC, NS = 2, 16
NW = NC * NS
BPW = BATCH // NW
CH = 128
NCH = BPW // CH
T_INV = 1.0 / 50.0
TC_ROWS = 4096


def _gather_body(user_hbm, item_hbm, theta_hbm, slip_hbm, guess_hbm,
                 thg_hbm, sraw_hbm, graw_hbm,
                 idx_u, idx_i, t0, t1, t2, t3, s0, s1, s2, s3,
                 g0, g1, g2, g3, sem_i0, sem_i1, sem_i2, sem_i3, sem_o):
    wid = lax.axis_index("s") * NC + lax.axis_index("c")
    base = wid * BPW

    pltpu.sync_copy(user_hbm.at[pl.ds(base, BPW)], idx_u)
    pltpu.sync_copy(item_hbm.at[pl.ds(base, BPW)], idx_i)

    tb, sb, gb = (t0, t1, t2, t3), (s0, s1, s2, s3), (g0, g1, g2, g3)
    sem_i = (sem_i0, sem_i1, sem_i2, sem_i3)

    def fire_in(ch):
        ids = pl.ds(ch * CH, CH)
        return (
            pltpu.async_copy(theta_hbm.at[idx_u.at[ids]], tb[ch], sem_i[ch]),
            pltpu.async_copy(slip_hbm.at[idx_i.at[ids]], sb[ch], sem_i[ch]),
            pltpu.async_copy(guess_hbm.at[idx_i.at[ids]], gb[ch], sem_i[ch]),
        )

    def fire_out(ch):
        ids = pl.ds(base + ch * CH, CH)
        return (
            pltpu.async_copy(tb[ch], thg_hbm.at[ids], sem_o),
            pltpu.async_copy(sb[ch], sraw_hbm.at[ids], sem_o),
            pltpu.async_copy(gb[ch], graw_hbm.at[ids], sem_o),
        )

    pend_in = [fire_in(ch) for ch in range(NCH)]
    pend_out = []
    for ch in range(NCH):
        for c in pend_in[ch]:
            c.wait()
        pend_out.append(fire_out(ch))
    for out_cp in pend_out:
        for c in out_cp:
            c.wait()


@jax.jit
def _sc_gather(user, item, theta_table, slip_flat, guess_flat):
    run = pl.kernel(
        _gather_body,
        out_type=(
            jax.ShapeDtypeStruct((BATCH, KNOW), jnp.float32),
            jax.ShapeDtypeStruct((BATCH,), jnp.float32),
            jax.ShapeDtypeStruct((BATCH,), jnp.float32),
        ),
        mesh=plsc.VectorSubcoreMesh(core_axis_name="c", subcore_axis_name="s",
                                    num_cores=NC, num_subcores=NS),
        compiler_params=pltpu.CompilerParams(needs_layout_passes=False),
        scratch_types=(
            [pltpu.VMEM((BPW,), jnp.int32)] * 2
            + [pltpu.VMEM((CH, KNOW), jnp.float32)] * NCH
            + [pltpu.VMEM((CH,), jnp.float32)] * NCH
            + [pltpu.VMEM((CH,), jnp.float32)] * NCH
            + [pltpu.SemaphoreType.DMA] * (NCH + 1)
        ),
        name="dina_sc_gather",
    )
    return run(user, item, theta_table, slip_flat, guess_flat)


def _combine_body(th_ref, kn_ref, sr_ref, gr_ref, o_ref):
    th = th_ref[...]
    kn = kn_ref[...]
    s = 0.5 * jnp.tanh(0.5 * th)
    prod_t = lax.transpose(kn * s, (1, 0))
    n = jnp.sum(prod_t, axis=0)
    p = 0.5 * jnp.tanh((0.5 * T_INV) * n) + 0.5
    slip = 0.2 * jnp.tanh(0.5 * sr_ref[...]) + 0.2
    guess = 0.2 * jnp.tanh(0.5 * gr_ref[...]) + 0.2
    o_ref[...] = guess + (1.0 - slip - guess) * p


@jax.jit
def _tc_combine(theta_g, knowledge, s_raw, g_raw):
    return pl.pallas_call(
        _combine_body,
        grid=(BATCH // TC_ROWS,),
        in_specs=[
            pl.BlockSpec((TC_ROWS, KNOW), lambda i: (i, 0)),
            pl.BlockSpec((TC_ROWS, KNOW), lambda i: (i, 0)),
            pl.BlockSpec((TC_ROWS,), lambda i: (i,)),
            pl.BlockSpec((TC_ROWS,), lambda i: (i,)),
        ],
        out_specs=pl.BlockSpec((TC_ROWS,), lambda i: (i,)),
        out_shape=jax.ShapeDtypeStruct((BATCH,), jnp.float32),
        name="dina_tc_combine",
    )(theta_g, knowledge, s_raw, g_raw)


def kernel(user, item, knowledge, theta_table, slip_table, guess_table):
    theta_g, s_raw, g_raw = _sc_gather(
        user, item, theta_table,
        slip_table.reshape(-1), guess_table.reshape(-1))
    return _tc_combine(theta_g, knowledge, s_raw, g_raw)
